# Initial kernel scaffold; baseline (speedup 1.0000x reference)
import jax
import jax.numpy as jnp
from jax.experimental import pallas as pl


def kernel(inputs, embedding_weight):
    raise NotImplementedError("write your pallas kernel here")


# speedup vs baseline: 1.6044x; 1.6044x over previous
import jax
import jax.numpy as jnp
from jax.experimental import pallas as pl

_NUM_EMBEDDINGS = 1024
_EMBEDDING_DIM = 64
_COMMITMENT_COST = 0.25
_BLK = 2048


def _vq_kernel(x_ref, emb_ref, q_ref, idx_ref, loss_ref):
    i = pl.program_id(0)
    x = x_ref[...]
    emb = emb_ref[...]
    x2 = jnp.sum(x ** 2, axis=1, keepdims=True)
    e2 = jnp.sum(emb ** 2, axis=1)
    mm = jax.lax.dot_general(
        x, emb, (((1,), (1,)), ((), ())),
        preferred_element_type=jnp.float32,
    )
    scores = x2 + e2[None, :] - 2.0 * mm

    min_s = jnp.min(scores, axis=1, keepdims=True)
    code_iota = jax.lax.broadcasted_iota(jnp.int32, scores.shape, 1)
    idx = jnp.min(
        jnp.where(scores == min_s, code_iota, _NUM_EMBEDDINGS), axis=1
    )
    idx_ref[0, 0, :] = idx

    onehot = (code_iota == idx[:, None]).astype(jnp.float32)
    q = jax.lax.dot_general(
        onehot, emb, (((1,), (0,)), ((), ())),
        preferred_element_type=jnp.float32,
    )
    q_ref[...] = q

    d = q - x
    part = jnp.sum(d * d).reshape(1, 1)

    @pl.when(i == 0)
    def _():
        loss_ref[...] = part

    @pl.when(i != 0)
    def _():
        loss_ref[...] += part


def kernel(inputs, embedding_weight):
    flat = inputs.reshape(-1, _EMBEDDING_DIM)
    n = flat.shape[0]
    nb = n // _BLK
    q, idx3, loss_acc = pl.pallas_call(
        _vq_kernel,
        grid=(nb,),
        in_specs=[
            pl.BlockSpec((_BLK, _EMBEDDING_DIM), lambda i: (i, 0)),
            pl.BlockSpec((_NUM_EMBEDDINGS, _EMBEDDING_DIM), lambda i: (0, 0)),
        ],
        out_specs=[
            pl.BlockSpec((_BLK, _EMBEDDING_DIM), lambda i: (i, 0)),
            pl.BlockSpec((1, 1, _BLK), lambda i: (i, 0, 0)),
            pl.BlockSpec((1, 1), lambda i: (0, 0)),
        ],
        out_shape=[
            jax.ShapeDtypeStruct((n, _EMBEDDING_DIM), jnp.float32),
            jax.ShapeDtypeStruct((nb, 1, _BLK), jnp.int32),
            jax.ShapeDtypeStruct((1, 1), jnp.float32),
        ],
    )(flat, embedding_weight)
    quantized = q.reshape(inputs.shape)
    loss = _COMMITMENT_COST * loss_acc[0, 0] / inputs.size
    encoding_indices = idx3.reshape(inputs.shape[:-1])
    return (quantized, loss, encoding_indices)


# speedup vs baseline: 1.6789x; 1.0464x over previous
import jax
import jax.numpy as jnp
from jax.experimental import pallas as pl

_NUM_EMBEDDINGS = 1024
_EMBEDDING_DIM = 64
_COMMITMENT_COST = 0.25
_BLK = 2048


def _vq_kernel(x_ref, emb_ref, q_ref, idx_ref, loss_ref):
    i = pl.program_id(0)
    x = x_ref[...]
    emb = emb_ref[...]
    x2 = jnp.sum(x ** 2, axis=1, keepdims=True)
    e2 = jnp.sum(emb ** 2, axis=1)
    mm = jax.lax.dot_general(
        x, emb, (((1,), (1,)), ((), ())),
        preferred_element_type=jnp.float32,
    )
    scores = x2 + e2[None, :] - 2.0 * mm

    idx = jnp.argmin(scores, axis=1).astype(jnp.int32)
    idx_ref[0, 0, :] = idx

    code_iota = jax.lax.broadcasted_iota(jnp.int32, scores.shape, 1)
    onehot = (code_iota == idx[:, None]).astype(jnp.float32)
    q = jax.lax.dot_general(
        onehot, emb, (((1,), (0,)), ((), ())),
        preferred_element_type=jnp.float32,
    )
    q_ref[...] = q

    d = q - x
    part = jnp.sum(d * d).reshape(1, 1)

    @pl.when(i == 0)
    def _():
        loss_ref[...] = part

    @pl.when(i != 0)
    def _():
        loss_ref[...] += part


def kernel(inputs, embedding_weight):
    flat = inputs.reshape(-1, _EMBEDDING_DIM)
    n = flat.shape[0]
    nb = n // _BLK
    q, idx3, loss_acc = pl.pallas_call(
        _vq_kernel,
        grid=(nb,),
        in_specs=[
            pl.BlockSpec((_BLK, _EMBEDDING_DIM), lambda i: (i, 0)),
            pl.BlockSpec((_NUM_EMBEDDINGS, _EMBEDDING_DIM), lambda i: (0, 0)),
        ],
        out_specs=[
            pl.BlockSpec((_BLK, _EMBEDDING_DIM), lambda i: (i, 0)),
            pl.BlockSpec((1, 1, _BLK), lambda i: (i, 0, 0)),
            pl.BlockSpec((1, 1), lambda i: (0, 0)),
        ],
        out_shape=[
            jax.ShapeDtypeStruct((n, _EMBEDDING_DIM), jnp.float32),
            jax.ShapeDtypeStruct((nb, 1, _BLK), jnp.int32),
            jax.ShapeDtypeStruct((1, 1), jnp.float32),
        ],
    )(flat, embedding_weight)
    quantized = q.reshape(inputs.shape)
    loss = _COMMITMENT_COST * loss_acc[0, 0] / inputs.size
    encoding_indices = idx3.reshape(inputs.shape[:-1])
    return (quantized, loss, encoding_indices)


# speedup vs baseline: 1.9772x; 1.1776x over previous
import jax
import jax.numpy as jnp
from jax.experimental import pallas as pl

_NUM_EMBEDDINGS = 1024
_EMBEDDING_DIM = 64
_COMMITMENT_COST = 0.25
_ROWS_PER_STEP = 8


def _vq_kernel(x_ref, emb_ref, q_ref, idx_ref, loss_ref):
    i = pl.program_id(0)
    blk = _ROWS_PER_STEP * x_ref.shape[1]
    x = x_ref[...].reshape(blk, _EMBEDDING_DIM)
    emb = emb_ref[...]
    x2 = jnp.sum(x ** 2, axis=1, keepdims=True)
    e2 = jnp.sum(emb ** 2, axis=1)
    mm = jax.lax.dot_general(
        x * -2.0, emb, (((1,), (1,)), ((), ())),
        preferred_element_type=jnp.float32,
    )
    scores = (x2 + e2[None, :]) + mm

    idx = jnp.argmin(scores, axis=1).astype(jnp.int32)
    idx_ref[...] = idx.reshape(_ROWS_PER_STEP, x_ref.shape[1])

    code_iota = jax.lax.broadcasted_iota(jnp.int32, scores.shape, 1)
    onehot = (code_iota == idx[:, None]).astype(jnp.float32)
    q = jax.lax.dot_general(
        onehot, emb, (((1,), (0,)), ((), ())),
        preferred_element_type=jnp.float32,
    )
    q_ref[...] = q.reshape(x_ref.shape)

    d = q - x
    part = jnp.sum(d * d).reshape(1, 1)

    @pl.when(i == 0)
    def _():
        loss_ref[...] = part

    @pl.when(i != 0)
    def _():
        loss_ref[...] += part


def kernel(inputs, embedding_weight):
    nmaj, nmin, _ = inputs.shape
    nb = nmaj // _ROWS_PER_STEP
    q, idx, loss_acc = pl.pallas_call(
        _vq_kernel,
        grid=(nb,),
        in_specs=[
            pl.BlockSpec((_ROWS_PER_STEP, nmin, _EMBEDDING_DIM),
                         lambda i: (i, 0, 0)),
            pl.BlockSpec((_NUM_EMBEDDINGS, _EMBEDDING_DIM), lambda i: (0, 0)),
        ],
        out_specs=[
            pl.BlockSpec((_ROWS_PER_STEP, nmin, _EMBEDDING_DIM),
                         lambda i: (i, 0, 0)),
            pl.BlockSpec((_ROWS_PER_STEP, nmin), lambda i: (i, 0)),
            pl.BlockSpec((1, 1), lambda i: (0, 0)),
        ],
        out_shape=[
            jax.ShapeDtypeStruct((nmaj, nmin, _EMBEDDING_DIM), jnp.float32),
            jax.ShapeDtypeStruct((nmaj, nmin), jnp.int32),
            jax.ShapeDtypeStruct((1, 1), jnp.float32),
        ],
    )(inputs, embedding_weight)
    loss = _COMMITMENT_COST * loss_acc[0, 0] / inputs.size
    return (q, loss, idx)
